# Initial kernel scaffold; baseline (speedup 1.0000x reference)
import jax
import jax.numpy as jnp
from jax.experimental import pallas as pl


def kernel(x, edge_index, edge_type, rel_emb, W, b):
    raise NotImplementedError("write your pallas kernel here")


# speedup vs baseline: 2.4719x; 2.4719x over previous
import functools

import jax
import jax.numpy as jnp
from jax import lax
from jax.experimental import pallas as pl
from jax.experimental.pallas import tpu as pltpu
from jax.experimental.pallas import tpu_sc as plsc

N = 10000
E = 320000
R = 16
F = 128
EMB = 64

NC = 2
NS = 16
L = 16

EP = E // NS
K = 128
NB = 160
EPAD = NB * K
NBUF = 2

NN = N // NC
RA = NN * R
DEAD = RA
ACC_ROWS = 80128
DSTRIPE = RA // NS
ZROWS = 128
NFC = F // L


_XB = 1000


def _proj_body(x_ref, w_ref, re_ref, b_ref, p1_ref, p2_ref, r3_ref):
    xb = x_ref[...]
    p1_ref[...] = jnp.dot(xb, w_ref[0:F, :], preferred_element_type=jnp.float32)
    p2_ref[...] = jnp.dot(xb, w_ref[F:2 * F, :], preferred_element_type=jnp.float32)

    @pl.when(pl.program_id(0) == 0)
    def _():
        r3_ref[...] = (
            jnp.dot(re_ref[...], w_ref[2 * F:, :], preferred_element_type=jnp.float32)
            + b_ref[...]
        )


def _proj(x, rel_emb, W, b2d):
    return pl.pallas_call(
        _proj_body,
        grid=(N // _XB,),
        in_specs=[
            pl.BlockSpec((_XB, F), lambda i: (i, 0)),
            pl.BlockSpec((2 * F + EMB, F), lambda i: (0, 0)),
            pl.BlockSpec((R, EMB), lambda i: (0, 0)),
            pl.BlockSpec((1, F), lambda i: (0, 0)),
        ],
        out_specs=[
            pl.BlockSpec((_XB, F), lambda i: (i, 0)),
            pl.BlockSpec((_XB, F), lambda i: (i, 0)),
            pl.BlockSpec((R, F), lambda i: (0, 0)),
        ],
        out_shape=[
            jax.ShapeDtypeStruct((N, F), jnp.float32),
            jax.ShapeDtypeStruct((N, F), jnp.float32),
            jax.ShapeDtypeStruct((R, F), jnp.float32),
        ],
    )(x, W, rel_emb, b2d)


def _sc_body(p1v, srch, dsth, typh, s2a, s2c,
             bigb, rid2, rb0, rb1, idxb0, idxb1, onesb, zb, accs,
             g0, g1):
    c = lax.axis_index("c")
    s = lax.axis_index("s")
    base = s * EP
    lo = c * NN
    rbufs = (rb0, rb1)
    idxbufs = (idxb0, idxb1)
    gsems = (g0, g1)

    pltpu.sync_copy(dsth.at[pl.ds(base, EP)], bigb.at[pl.ds(0, EP)])

    def rid_step(i, carry):
        d = bigb[pl.ds(i * L, L)]
        dl = d - lo
        m = (dl >= 0) & (dl < NN)
        ridv = jnp.where(m, dl * R, DEAD)
        rid2[i // 8, pl.ds((i % 8) * L, L)] = ridv
        return carry

    lax.fori_loop(0, EP // L, rid_step, 0)

    pltpu.sync_copy(typh.at[pl.ds(base, EP)], bigb.at[pl.ds(0, EP)])

    def rid_add(i, carry):
        t = bigb[pl.ds(i * L, L)]
        r0 = rid2[i // 8, pl.ds((i % 8) * L, L)]
        rid2[i // 8, pl.ds((i % 8) * L, L)] = r0 + t
        return carry

    lax.fori_loop(0, EP // L, rid_add, 0)

    def rid_tail(i, carry):
        f = EP + i * L
        rid2[f // K, pl.ds(f % K, L)] = jnp.full((L,), DEAD, jnp.int32)
        return carry

    lax.fori_loop(0, (EPAD - EP) // L, rid_tail, 0)

    pltpu.sync_copy(srch.at[pl.ds(base, EP)], bigb.at[pl.ds(0, EP)])

    def idx_tail(i, carry):
        bigb[pl.ds(EP + i * L, L)] = jnp.zeros((L,), jnp.int32)
        return carry

    lax.fori_loop(0, (EPAD - EP) // L, idx_tail, 0)

    def idx_scale(i, carry):
        bigb[pl.ds(i * L, L)] = bigb[pl.ds(i * L, L)] * NFC
        return carry

    lax.fori_loop(0, EPAD // L, idx_scale, 0)

    def ones_fill(i, carry):
        onesb[i, :] = jnp.full((L,), 1.0, jnp.float32)
        return carry

    lax.fori_loop(0, K, ones_fill, 0)

    def zero_fill(i, carry):
        zb[i, :] = jnp.zeros((L,), jnp.float32)
        return carry

    lax.fori_loop(0, ZROWS, zero_fill, 0)

    nzc = jnp.where(s == NS - 1, 626 - 40 * (NS - 1), 40)

    def _zero_stripe():
        def zs(i, carry):
            pltpu.sync_copy(
                zb, accs.at[pl.ds((s * 40 + i) * ZROWS, ZROWS)])
            return carry

        lax.fori_loop(0, nzc, zs, 0)

    _zero_stripe()
    plsc.subcore_barrier()

    def _make_idx_and_gather(j, bslot, fc):
        ib = idxbufs[bslot]

        def mk(i, carry):
            ib[pl.ds(i * L, L)] = bigb[pl.ds(j * K + i * L, L)] + fc
            return carry

        lax.fori_loop(0, K // L, mk, 0)
        pltpu.async_copy(p1v.at[ib], rbufs[bslot], gsems[bslot])

    def _gather_wait(bslot):
        pltpu.make_async_copy(
            p1v.at[idxbufs[bslot]], rbufs[bslot], gsems[bslot]
        ).wait()

    for fc in range(NFC):
        for bslot in range(NBUF):
            _make_idx_and_gather(bslot, bslot, fc)

        def pass_body(o, carry):
            for kk in range(NBUF):
                j = o * NBUF + kk
                _gather_wait(kk)
                pltpu.sync_copy(rbufs[kk], accs.at[rid2.at[j]], add=True)
                jn = j + NBUF

                @pl.when(jn < NB)
                def _():
                    _make_idx_and_gather(jn, kk, fc)
            return carry

        lax.fori_loop(0, NB // NBUF, pass_body, 0)
        plsc.subcore_barrier()

        pltpu.sync_copy(
            accs.at[pl.ds(s * DSTRIPE, DSTRIPE)],
            s2a.at[pl.ds(c * RA + s * DSTRIPE, DSTRIPE), pl.ds(fc * L, L)],
        )
        plsc.subcore_barrier()
        _zero_stripe()
        plsc.subcore_barrier()

    def cnt_body(o, carry):
        for kk in range(NBUF):
            j = o * NBUF + kk
            pltpu.sync_copy(onesb, accs.at[rid2.at[j]], add=True)
        return carry

    lax.fori_loop(0, NB // NBUF, cnt_body, 0)
    plsc.subcore_barrier()
    pltpu.sync_copy(
        accs.at[pl.ds(s * DSTRIPE, DSTRIPE)],
        s2c.at[pl.ds(c * RA + s * DSTRIPE, DSTRIPE)],
    )


_sc_call = functools.partial(
    pl.kernel,
    out_type=[
        jax.ShapeDtypeStruct((N * R, F), jnp.float32),
        jax.ShapeDtypeStruct((N * R, L), jnp.float32),
    ],
    mesh=plsc.VectorSubcoreMesh(core_axis_name="c", subcore_axis_name="s"),
    scratch_types=[
        pltpu.VMEM((EPAD,), jnp.int32),
        pltpu.VMEM((NB, K), jnp.int32),
        pltpu.VMEM((K, L), jnp.float32),
        pltpu.VMEM((K, L), jnp.float32),
        pltpu.VMEM((K,), jnp.int32),
        pltpu.VMEM((K,), jnp.int32),
        pltpu.VMEM((K, L), jnp.float32),
        pltpu.VMEM((ZROWS, L), jnp.float32),
        pltpu.VMEM_SHARED((ACC_ROWS, L), jnp.float32),
        pltpu.SemaphoreType.DMA,
        pltpu.SemaphoreType.DMA,
    ],
    compiler_params=pltpu.CompilerParams(use_tc_tiling_on_sc=False),
)(_sc_body)


_FB = 400


def _fin_body(s_ref, c_ref, p2_ref, r3_ref, o_ref):
    sv = s_ref[...]
    cnt = c_ref[:, :, 0:1]
    p2 = p2_ref[...][:, None, :]
    r3 = r3_ref[...][None, :, :]
    agg = sv + cnt * (p2 + r3)
    o_ref[...] = jnp.tanh(agg).sum(axis=1)


def _finalize(s3, c3, P2, r3):
    return pl.pallas_call(
        _fin_body,
        grid=(N // _FB,),
        in_specs=[
            pl.BlockSpec((_FB, R, F), lambda i: (i, 0, 0)),
            pl.BlockSpec((_FB, R, L), lambda i: (i, 0, 0)),
            pl.BlockSpec((_FB, F), lambda i: (i, 0)),
            pl.BlockSpec((R, F), lambda i: (0, 0)),
        ],
        out_specs=pl.BlockSpec((_FB, F), lambda i: (i, 0)),
        out_shape=jax.ShapeDtypeStruct((N, F), jnp.float32),
    )(s3, c3, P2, r3)


def kernel(x, edge_index, edge_type, rel_emb, W, b):
    P1, P2, r3 = _proj(x, rel_emb, W, b.reshape(1, F))
    p1v = P1.reshape(N * NFC, L)
    src = edge_index[0].astype(jnp.int32)
    dst = edge_index[1].astype(jnp.int32)
    typ = edge_type.astype(jnp.int32)
    s2a, s2c = _sc_call(p1v, src, dst, typ)
    return _finalize(
        s2a.reshape(N, R, F), s2c.reshape(N, R, L), P2, r3
    )


# speedup vs baseline: 2.8010x; 1.1331x over previous
import functools

import jax
import jax.numpy as jnp
from jax import lax
from jax.experimental import pallas as pl
from jax.experimental.pallas import tpu as pltpu
from jax.experimental.pallas import tpu_sc as plsc

N = 10000
E = 320000
R = 16
F = 128
EMB = 64

NC = 2
NS = 16
L = 16

EP = E // NS
K = 128
NB = 160
EPAD = NB * K
NBUF = 4

NN = N // NC
RA = NN * R
DEAD = RA
ACC_ROWS = 80128
DSTRIPE = RA // NS
ZROWS = 64
ZCH = 79
NFC = F // L


_XB = 1000


def _proj_body(x_ref, w_ref, re_ref, b_ref, p1_ref, p2_ref, r3_ref):
    xb = x_ref[...]
    p1_ref[...] = jnp.dot(xb, w_ref[0:F, :], preferred_element_type=jnp.float32)
    p2_ref[...] = jnp.dot(xb, w_ref[F:2 * F, :], preferred_element_type=jnp.float32)

    @pl.when(pl.program_id(0) == 0)
    def _():
        r3_ref[...] = (
            jnp.dot(re_ref[...], w_ref[2 * F:, :], preferred_element_type=jnp.float32)
            + b_ref[...]
        )


def _proj(x, rel_emb, W, b2d):
    return pl.pallas_call(
        _proj_body,
        grid=(N // _XB,),
        in_specs=[
            pl.BlockSpec((_XB, F), lambda i: (i, 0)),
            pl.BlockSpec((2 * F + EMB, F), lambda i: (0, 0)),
            pl.BlockSpec((R, EMB), lambda i: (0, 0)),
            pl.BlockSpec((1, F), lambda i: (0, 0)),
        ],
        out_specs=[
            pl.BlockSpec((_XB, F), lambda i: (i, 0)),
            pl.BlockSpec((_XB, F), lambda i: (i, 0)),
            pl.BlockSpec((R, F), lambda i: (0, 0)),
        ],
        out_shape=[
            jax.ShapeDtypeStruct((N, F), jnp.float32),
            jax.ShapeDtypeStruct((N, F), jnp.float32),
            jax.ShapeDtypeStruct((R, F), jnp.float32),
        ],
    )(x, W, rel_emb, b2d)


def _sc_body(p1v, srch, dsth, typh, s2a, s2c,
             bigb, rid2, rb0, rb1, rb2, rb3, zb, accs,
             g0, g1, g2, g3, s0, s1, s2, s3):
    c = lax.axis_index("c")
    s = lax.axis_index("s")
    base = s * EP
    lo = c * NN
    rbufs = (rb0, rb1, rb2, rb3)
    gsems = (g0, g1, g2, g3)
    ssems = (s0, s1, s2, s3)

    pltpu.sync_copy(dsth.at[pl.ds(base, EP)], bigb.at[pl.ds(0, EP)])

    def rid_step(i, carry):
        d = bigb[pl.ds(i * L, L)]
        dl = d - lo
        m = (dl >= 0) & (dl < NN)
        ridv = jnp.where(m, dl * R, DEAD)
        rid2[i // 8, pl.ds((i % 8) * L, L)] = ridv
        return carry

    lax.fori_loop(0, EP // L, rid_step, 0)

    pltpu.sync_copy(typh.at[pl.ds(base, EP)], bigb.at[pl.ds(0, EP)])

    def rid_add(i, carry):
        t = bigb[pl.ds(i * L, L)]
        r0 = rid2[i // 8, pl.ds((i % 8) * L, L)]
        rid2[i // 8, pl.ds((i % 8) * L, L)] = r0 + t
        return carry

    lax.fori_loop(0, EP // L, rid_add, 0)

    def rid_tail(i, carry):
        f = EP + i * L
        rid2[f // K, pl.ds(f % K, L)] = jnp.full((L,), DEAD, jnp.int32)
        return carry

    lax.fori_loop(0, (EPAD - EP) // L, rid_tail, 0)

    pltpu.sync_copy(srch.at[pl.ds(base, EP)], bigb.at[pl.ds(0, EP)])

    def idx_tail(i, carry):
        bigb[pl.ds(EP + i * L, L)] = jnp.zeros((L,), jnp.int32)
        return carry

    lax.fori_loop(0, (EPAD - EP) // L, idx_tail, 0)

    def idx_scale(i, carry):
        bigb[pl.ds(i * L, L)] = bigb[pl.ds(i * L, L)] * NFC
        return carry

    lax.fori_loop(0, EPAD // L, idx_scale, 0)

    def zero_fill(i, carry):
        zb[i, :] = jnp.zeros((L,), jnp.float32)
        return carry

    lax.fori_loop(0, ZROWS, zero_fill, 0)

    nzc = jnp.where(s == NS - 1, ACC_ROWS // ZROWS - ZCH * (NS - 1), ZCH)

    def _zero_stripe():
        def zs(i, carry):
            pltpu.sync_copy(
                zb, accs.at[pl.ds((s * ZCH + i) * ZROWS, ZROWS)])
            return carry

        lax.fori_loop(0, nzc, zs, 0)

    _zero_stripe()
    plsc.subcore_barrier()

    def _gather(j, bslot):
        pltpu.async_copy(
            p1v.at[bigb.at[pl.ds(j * K, K)]], rbufs[bslot], gsems[bslot])

    def _gather_wait(j, bslot):
        pltpu.make_async_copy(
            p1v.at[bigb.at[pl.ds(j * K, K)]], rbufs[bslot], gsems[bslot]
        ).wait()

    def _scatter(j, bslot):
        pltpu.async_copy(
            rbufs[bslot], accs.at[rid2.at[j]], ssems[bslot], add=True)

    def _scatter_wait(j, bslot):
        pltpu.make_async_copy(
            rbufs[bslot], accs.at[rid2.at[j]], ssems[bslot]
        ).wait()

    for fc in range(NFC):
        for bslot in range(2):
            _gather(bslot, bslot)

        def pass_body(o, carry):
            for kk in range(NBUF):
                j = o * NBUF + kk
                bp = (kk + 2) % NBUF

                @pl.when(j >= 2)
                def _():
                    _scatter_wait(j - 2, bp)

                @pl.when(j + 2 < NB)
                def _():
                    _gather(j + 2, bp)

                _gather_wait(j, kk)
                _scatter(j, kk)
            return carry

        lax.fori_loop(0, NB // NBUF, pass_body, 0)
        _scatter_wait(NB - 2, (NB - 2) % NBUF)
        _scatter_wait(NB - 1, (NB - 1) % NBUF)
        plsc.subcore_barrier()

        pltpu.sync_copy(
            accs.at[pl.ds(s * DSTRIPE, DSTRIPE)],
            s2a.at[pl.ds(c * RA + s * DSTRIPE, DSTRIPE), pl.ds(fc * L, L)],
        )
        plsc.subcore_barrier()
        _zero_stripe()

        if fc < NFC - 1:
            def idx_inc(i, carry):
                bigb[pl.ds(i * L, L)] = bigb[pl.ds(i * L, L)] + 1
                return carry

            lax.fori_loop(0, EPAD // L, idx_inc, 0)
        plsc.subcore_barrier()

    def ones_fill(i, carry):
        rb0[i, :] = jnp.full((L,), 1.0, jnp.float32)
        return carry

    lax.fori_loop(0, K, ones_fill, 0)

    def cnt_body(o, carry):
        for kk in range(NBUF):
            j = o * NBUF + kk

            @pl.when(j >= NBUF)
            def _():
                pltpu.make_async_copy(
                    rb0, accs.at[rid2.at[j - NBUF]], ssems[kk]).wait()

            pltpu.async_copy(rb0, accs.at[rid2.at[j]], ssems[kk], add=True)
        return carry

    lax.fori_loop(0, NB // NBUF, cnt_body, 0)
    for kk in range(NBUF):
        pltpu.make_async_copy(
            rb0, accs.at[rid2.at[NB - NBUF + kk]], ssems[kk]).wait()
    plsc.subcore_barrier()
    pltpu.sync_copy(
        accs.at[pl.ds(s * DSTRIPE, DSTRIPE)],
        s2c.at[pl.ds(c * RA + s * DSTRIPE, DSTRIPE)],
    )


_sc_call = functools.partial(
    pl.kernel,
    out_type=[
        jax.ShapeDtypeStruct((N * R, F), jnp.float32),
        jax.ShapeDtypeStruct((N * R, L), jnp.float32),
    ],
    mesh=plsc.VectorSubcoreMesh(core_axis_name="c", subcore_axis_name="s"),
    scratch_types=[
        pltpu.VMEM((EPAD,), jnp.int32),
        pltpu.VMEM((NB, K), jnp.int32),
        pltpu.VMEM((K, L), jnp.float32),
        pltpu.VMEM((K, L), jnp.float32),
        pltpu.VMEM((K, L), jnp.float32),
        pltpu.VMEM((K, L), jnp.float32),
        pltpu.VMEM((ZROWS, L), jnp.float32),
        pltpu.VMEM_SHARED((ACC_ROWS, L), jnp.float32),
        pltpu.SemaphoreType.DMA,
        pltpu.SemaphoreType.DMA,
        pltpu.SemaphoreType.DMA,
        pltpu.SemaphoreType.DMA,
        pltpu.SemaphoreType.DMA,
        pltpu.SemaphoreType.DMA,
        pltpu.SemaphoreType.DMA,
        pltpu.SemaphoreType.DMA,
    ],
    compiler_params=pltpu.CompilerParams(use_tc_tiling_on_sc=False),
)(_sc_body)


_FB = 400


def _fin_body(s_ref, c_ref, p2_ref, r3_ref, o_ref):
    sv = s_ref[...]
    cnt = c_ref[:, :, 0:1]
    p2 = p2_ref[...][:, None, :]
    r3 = r3_ref[...][None, :, :]
    agg = sv + cnt * (p2 + r3)
    o_ref[...] = jnp.tanh(agg).sum(axis=1)


def _finalize(s3, c3, P2, r3):
    return pl.pallas_call(
        _fin_body,
        grid=(N // _FB,),
        in_specs=[
            pl.BlockSpec((_FB, R, F), lambda i: (i, 0, 0)),
            pl.BlockSpec((_FB, R, L), lambda i: (i, 0, 0)),
            pl.BlockSpec((_FB, F), lambda i: (i, 0)),
            pl.BlockSpec((R, F), lambda i: (0, 0)),
        ],
        out_specs=pl.BlockSpec((_FB, F), lambda i: (i, 0)),
        out_shape=jax.ShapeDtypeStruct((N, F), jnp.float32),
    )(s3, c3, P2, r3)


def kernel(x, edge_index, edge_type, rel_emb, W, b):
    P1, P2, r3 = _proj(x, rel_emb, W, b.reshape(1, F))
    p1v = P1.reshape(N * NFC, L)
    src = edge_index[0].astype(jnp.int32)
    dst = edge_index[1].astype(jnp.int32)
    typ = edge_type.astype(jnp.int32)
    s2a, s2c = _sc_call(p1v, src, dst, typ)
    return _finalize(
        s2a.reshape(N, R, F), s2c.reshape(N, R, L), P2, r3
    )


# speedup vs baseline: 4.1319x; 1.4752x over previous
import functools

import jax
import jax.numpy as jnp
from jax import lax
from jax.experimental import pallas as pl
from jax.experimental.pallas import tpu as pltpu
from jax.experimental.pallas import tpu_sc as plsc

N = 10000
E = 320000
R = 16
F = 128
EMB = 64

NC = 2
NS = 16
L = 16

EP = E // NS
K = 128
NB = 160
EPAD = NB * K
NBUF = 4

NN = N // NC
RA = NN * R
DEAD = RA
ACC_ROWS = 80128
DSTRIPE = RA // NS
ZROWS = 64
ZCH = 79
NFC = F // L
WN = 2000
PK = 1 << 17
PBUF = EPAD + 528


_XB = 1000


def _proj_body(x_ref, w_ref, re_ref, b_ref, p1_ref, p2_ref, r3_ref):
    xb = x_ref[...]
    p1_ref[...] = jnp.dot(xb, w_ref[0:F, :], preferred_element_type=jnp.float32)
    p2_ref[...] = jnp.dot(xb, w_ref[F:2 * F, :], preferred_element_type=jnp.float32)

    @pl.when(pl.program_id(0) == 0)
    def _():
        r3_ref[...] = (
            jnp.dot(re_ref[...], w_ref[2 * F:, :], preferred_element_type=jnp.float32)
            + b_ref[...]
        )


def _proj(x, rel_emb, W, b2d):
    return pl.pallas_call(
        _proj_body,
        grid=(N // _XB,),
        in_specs=[
            pl.BlockSpec((_XB, F), lambda i: (i, 0)),
            pl.BlockSpec((2 * F + EMB, F), lambda i: (0, 0)),
            pl.BlockSpec((R, EMB), lambda i: (0, 0)),
            pl.BlockSpec((1, F), lambda i: (0, 0)),
        ],
        out_specs=[
            pl.BlockSpec((_XB, F), lambda i: (i, 0)),
            pl.BlockSpec((_XB, F), lambda i: (i, 0)),
            pl.BlockSpec((R, F), lambda i: (0, 0)),
        ],
        out_shape=[
            jax.ShapeDtypeStruct((N, F), jnp.float32),
            jax.ShapeDtypeStruct((N, F), jnp.float32),
            jax.ShapeDtypeStruct((R, F), jnp.float32),
        ],
    )(x, W, rel_emb, b2d)


def _sc_body(p1v, srch, dsth, typh, s2a, s2c,
             packedb, dstw, typw, srcw,
             ridd0, ridd1, ridd2, ridd3, idxd0, idxd1, idxd2, idxd3,
             rb0, rb1, rb2, rb3, zb, accs,
             g0, g1, g2, g3, s0, s1, s2, s3):
    c = lax.axis_index("c")
    s = lax.axis_index("s")
    base = s * EP
    lo = c * NN
    rbufs = (rb0, rb1, rb2, rb3)
    ridd = (ridd0, ridd1, ridd2, ridd3)
    idxd = (idxd0, idxd1, idxd2, idxd3)
    gsems = (g0, g1, g2, g3)
    ssems = (s0, s1, s2, s3)

    off = jnp.int32(0)
    for w in range(EP // WN):
        wb = base + w * WN
        pltpu.sync_copy(dsth.at[pl.ds(wb, WN)], dstw)
        pltpu.sync_copy(typh.at[pl.ds(wb, WN)], typw)
        pltpu.sync_copy(srch.at[pl.ds(wb, WN)], srcw)

        def cstep(i, o):
            d = dstw[pl.ds(i * L, L)]
            t = typw[pl.ds(i * L, L)]
            sv = srcw[pl.ds(i * L, L)]
            dl = d - lo
            m = (dl >= 0) & (dl < NN)
            packed = sv * PK + (dl * R + t)
            cs = plsc.cumsum(jnp.where(m, jnp.int32(1), jnp.int32(0)))
            plsc.store_scatter(packedb, [o + cs - 1], packed, mask=m)
            return o + cs[L - 1]

        off = lax.fori_loop(0, WN // L, cstep, off)

    nbat = jnp.maximum((off + (K - 1)) // K, 4)
    nbat4 = ((nbat + 3) // 4) * 4
    npadg = (nbat4 * K - off + (L - 1)) // L

    def pstep(i, carry):
        packedb[pl.ds(off + i * L, L)] = jnp.full((L,), DEAD, jnp.int32)
        return carry

    lax.fori_loop(0, npadg, pstep, 0)

    def zero_fill(i, carry):
        zb[i, :] = jnp.zeros((L,), jnp.float32)
        return carry

    lax.fori_loop(0, ZROWS, zero_fill, 0)

    nzc = jnp.where(s == NS - 1, ACC_ROWS // ZROWS - ZCH * (NS - 1), ZCH)

    def _zero_stripe():
        def zs(i, carry):
            pltpu.sync_copy(
                zb, accs.at[pl.ds((s * ZCH + i) * ZROWS, ZROWS)])
            return carry

        lax.fori_loop(0, nzc, zs, 0)

    _zero_stripe()
    plsc.subcore_barrier()

    def _decode(j, bslot, fc, with_idx):
        def dk(i, carry):
            v = packedb[pl.ds(j * K + i * L, L)]
            ridd[bslot][pl.ds(i * L, L)] = v & (PK - 1)
            if with_idx:
                idxd[bslot][pl.ds(i * L, L)] = (v // PK) * NFC + fc
            return carry

        lax.fori_loop(0, K // L, dk, 0)

    def _gather(bslot):
        pltpu.async_copy(p1v.at[idxd[bslot]], rbufs[bslot], gsems[bslot])

    def _gather_wait(bslot):
        pltpu.make_async_copy(
            p1v.at[idxd[bslot]], rbufs[bslot], gsems[bslot]).wait()

    def _scatter(bslot):
        pltpu.async_copy(
            rbufs[bslot], accs.at[ridd[bslot]], ssems[bslot], add=True)

    def _scatter_wait(bslot):
        pltpu.make_async_copy(
            rbufs[bslot], accs.at[ridd[bslot]], ssems[bslot]).wait()

    for fc in range(NFC):
        for bslot in range(2):
            _decode(bslot, bslot, fc, True)
            _gather(bslot)

        def pass_body(o, carry):
            for kk in range(NBUF):
                j = o * NBUF + kk
                bp = (kk + 2) % NBUF

                @pl.when(j >= 2)
                def _():
                    _scatter_wait(bp)

                @pl.when(j + 2 < nbat4)
                def _():
                    _decode(j + 2, bp, fc, True)
                    _gather(bp)

                _gather_wait(kk)
                _scatter(kk)
            return carry

        lax.fori_loop(0, nbat4 // 4, pass_body, 0)
        _scatter_wait(2)
        _scatter_wait(3)
        plsc.subcore_barrier()

        pltpu.sync_copy(
            accs.at[pl.ds(s * DSTRIPE, DSTRIPE)],
            s2a.at[pl.ds(c * RA + s * DSTRIPE, DSTRIPE), pl.ds(fc * L, L)],
        )
        plsc.subcore_barrier()
        _zero_stripe()
        plsc.subcore_barrier()

    def ones_fill(i, carry):
        rb0[i, :] = jnp.full((L,), 1.0, jnp.float32)
        return carry

    lax.fori_loop(0, K, ones_fill, 0)

    def _cscatter(bslot):
        pltpu.async_copy(rb0, accs.at[ridd[bslot]], ssems[bslot], add=True)

    def _cscatter_wait(bslot):
        pltpu.make_async_copy(rb0, accs.at[ridd[bslot]], ssems[bslot]).wait()

    def cnt_body(o, carry):
        for kk in range(NBUF):
            j = o * NBUF + kk

            @pl.when(j >= NBUF)
            def _():
                _cscatter_wait(kk)

            _decode(j, kk, 0, False)
            _cscatter(kk)
        return carry

    lax.fori_loop(0, nbat4 // 4, cnt_body, 0)
    for kk in range(NBUF):
        _cscatter_wait(kk)
    plsc.subcore_barrier()
    pltpu.sync_copy(
        accs.at[pl.ds(s * DSTRIPE, DSTRIPE)],
        s2c.at[pl.ds(c * RA + s * DSTRIPE, DSTRIPE)],
    )


_sc_call = functools.partial(
    pl.kernel,
    out_type=[
        jax.ShapeDtypeStruct((N * R, F), jnp.float32),
        jax.ShapeDtypeStruct((N * R, L), jnp.float32),
    ],
    mesh=plsc.VectorSubcoreMesh(core_axis_name="c", subcore_axis_name="s"),
    scratch_types=[
        pltpu.VMEM((PBUF,), jnp.int32),
        pltpu.VMEM((WN,), jnp.int32),
        pltpu.VMEM((WN,), jnp.int32),
        pltpu.VMEM((WN,), jnp.int32),
        pltpu.VMEM((K,), jnp.int32),
        pltpu.VMEM((K,), jnp.int32),
        pltpu.VMEM((K,), jnp.int32),
        pltpu.VMEM((K,), jnp.int32),
        pltpu.VMEM((K,), jnp.int32),
        pltpu.VMEM((K,), jnp.int32),
        pltpu.VMEM((K,), jnp.int32),
        pltpu.VMEM((K,), jnp.int32),
        pltpu.VMEM((K, L), jnp.float32),
        pltpu.VMEM((K, L), jnp.float32),
        pltpu.VMEM((K, L), jnp.float32),
        pltpu.VMEM((K, L), jnp.float32),
        pltpu.VMEM((ZROWS, L), jnp.float32),
        pltpu.VMEM_SHARED((ACC_ROWS, L), jnp.float32),
        pltpu.SemaphoreType.DMA,
        pltpu.SemaphoreType.DMA,
        pltpu.SemaphoreType.DMA,
        pltpu.SemaphoreType.DMA,
        pltpu.SemaphoreType.DMA,
        pltpu.SemaphoreType.DMA,
        pltpu.SemaphoreType.DMA,
        pltpu.SemaphoreType.DMA,
    ],
    compiler_params=pltpu.CompilerParams(
        use_tc_tiling_on_sc=False, needs_layout_passes=False),
)(_sc_body)


_FB = 400


def _fin_body(s_ref, c_ref, p2_ref, r3_ref, o_ref):
    sv = s_ref[...]
    cnt = c_ref[:, :, 0:1]
    p2 = p2_ref[...][:, None, :]
    r3 = r3_ref[...][None, :, :]
    agg = sv + cnt * (p2 + r3)
    o_ref[...] = jnp.tanh(agg).sum(axis=1)


def _finalize(s3, c3, P2, r3):
    return pl.pallas_call(
        _fin_body,
        grid=(N // _FB,),
        in_specs=[
            pl.BlockSpec((_FB, R, F), lambda i: (i, 0, 0)),
            pl.BlockSpec((_FB, R, L), lambda i: (i, 0, 0)),
            pl.BlockSpec((_FB, F), lambda i: (i, 0)),
            pl.BlockSpec((R, F), lambda i: (0, 0)),
        ],
        out_specs=pl.BlockSpec((_FB, F), lambda i: (i, 0)),
        out_shape=jax.ShapeDtypeStruct((N, F), jnp.float32),
    )(s3, c3, P2, r3)


def kernel(x, edge_index, edge_type, rel_emb, W, b):
    P1, P2, r3 = _proj(x, rel_emb, W, b.reshape(1, F))
    p1v = P1.reshape(N * NFC, L)
    src = edge_index[0].astype(jnp.int32)
    dst = edge_index[1].astype(jnp.int32)
    typ = edge_type.astype(jnp.int32)
    s2a, s2c = _sc_call(p1v, src, dst, typ)
    return _finalize(
        s2a.reshape(N, R, F), s2c.reshape(N, R, L), P2, r3
    )


# speedup vs baseline: 4.3238x; 1.0464x over previous
import functools

import jax
import jax.numpy as jnp
from jax import lax
from jax.experimental import pallas as pl
from jax.experimental.pallas import tpu as pltpu
from jax.experimental.pallas import tpu_sc as plsc

N = 10000
E = 320000
R = 16
F = 128
EMB = 64

NC = 2
NS = 16
L = 16

EP = E // NS
K = 128
NB = 160
EPAD = NB * K
NBUF = 4

NN = N // NC
RA = NN * R
DEAD = RA
ACC_ROWS = 80128
DSTRIPE = RA // NS
ZROWS = 500
NFC = F // L
WN = 2000
PK = 1 << 17
PBUF = EPAD + 528


_XB = 1000


def _proj_body(x_ref, w_ref, re_ref, b_ref, p1_ref, p2_ref, r3_ref):
    xb = x_ref[...]
    p1_ref[...] = jnp.dot(xb, w_ref[0:F, :], preferred_element_type=jnp.float32)
    p2_ref[...] = jnp.dot(xb, w_ref[F:2 * F, :], preferred_element_type=jnp.float32)

    @pl.when(pl.program_id(0) == 0)
    def _():
        r3_ref[...] = (
            jnp.dot(re_ref[...], w_ref[2 * F:, :], preferred_element_type=jnp.float32)
            + b_ref[...]
        )


def _proj(x, rel_emb, W, b2d):
    return pl.pallas_call(
        _proj_body,
        grid=(N // _XB,),
        in_specs=[
            pl.BlockSpec((_XB, F), lambda i: (i, 0)),
            pl.BlockSpec((2 * F + EMB, F), lambda i: (0, 0)),
            pl.BlockSpec((R, EMB), lambda i: (0, 0)),
            pl.BlockSpec((1, F), lambda i: (0, 0)),
        ],
        out_specs=[
            pl.BlockSpec((_XB, F), lambda i: (i, 0)),
            pl.BlockSpec((_XB, F), lambda i: (i, 0)),
            pl.BlockSpec((R, F), lambda i: (0, 0)),
        ],
        out_shape=[
            jax.ShapeDtypeStruct((N, F), jnp.float32),
            jax.ShapeDtypeStruct((N, F), jnp.float32),
            jax.ShapeDtypeStruct((R, F), jnp.float32),
        ],
    )(x, W, rel_emb, b2d)


def _sc_body(p1v, srch, dsth, typh, s2a, s2c,
             packedb, dstw, typw, srcw,
             ridd0, ridd1, ridd2, ridd3, idxd0, idxd1, idxd2, idxd3,
             rb0, rb1, rb2, rb3, zb, accs,
             g0, g1, g2, g3, s0, s1, s2, s3):
    c = lax.axis_index("c")
    s = lax.axis_index("s")
    base = s * EP
    lo = c * NN
    rbufs = (rb0, rb1, rb2, rb3)
    ridd = (ridd0, ridd1, ridd2, ridd3)
    idxd = (idxd0, idxd1, idxd2, idxd3)
    gsems = (g0, g1, g2, g3)
    ssems = (s0, s1, s2, s3)

    off = jnp.int32(0)
    for w in range(EP // WN):
        wb = base + w * WN
        pltpu.async_copy(dsth.at[pl.ds(wb, WN)], dstw, g0)
        pltpu.async_copy(typh.at[pl.ds(wb, WN)], typw, g1)
        pltpu.async_copy(srch.at[pl.ds(wb, WN)], srcw, g2)
        pltpu.make_async_copy(dsth.at[pl.ds(wb, WN)], dstw, g0).wait()
        pltpu.make_async_copy(typh.at[pl.ds(wb, WN)], typw, g1).wait()
        pltpu.make_async_copy(srch.at[pl.ds(wb, WN)], srcw, g2).wait()

        def cstep(i, o):
            d = dstw[pl.ds(i * L, L)]
            t = typw[pl.ds(i * L, L)]
            sv = srcw[pl.ds(i * L, L)]
            dl = d - lo
            m = (dl >= 0) & (dl < NN)
            packed = sv * PK + (dl * R + t)
            cs = plsc.cumsum(jnp.where(m, jnp.int32(1), jnp.int32(0)))
            plsc.store_scatter(packedb, [o + cs - 1], packed, mask=m)
            return o + cs[L - 1]

        off = lax.fori_loop(0, WN // L, cstep, off)

    nbat = jnp.maximum((off + (K - 1)) // K, 4)
    nbat4 = ((nbat + 3) // 4) * 4
    npadg = (nbat4 * K - off + (L - 1)) // L

    def pstep(i, carry):
        packedb[pl.ds(off + i * L, L)] = jnp.full((L,), DEAD, jnp.int32)
        return carry

    lax.fori_loop(0, npadg, pstep, 0)

    def zero_fill(i, carry):
        zb[i, :] = jnp.zeros((L,), jnp.float32)
        return carry

    lax.fori_loop(0, ZROWS, zero_fill, 0)

    def _zero_stripe():
        for kk in range(DSTRIPE // ZROWS):
            pltpu.sync_copy(
                zb, accs.at[pl.ds(s * DSTRIPE + kk * ZROWS, ZROWS)])
        pltpu.sync_copy(zb.at[pl.ds(0, 8)], accs.at[pl.ds(DEAD + s * 8, 8)])

    _zero_stripe()
    plsc.subcore_barrier()

    def _decode(j, bslot, fc, with_idx):
        def dk(i, carry):
            v = packedb[pl.ds(j * K + i * L, L)]
            ridd[bslot][pl.ds(i * L, L)] = v & (PK - 1)
            if with_idx:
                idxd[bslot][pl.ds(i * L, L)] = (v // PK) * NFC + fc
            return carry

        lax.fori_loop(0, K // L, dk, 0)

    def _gather(bslot):
        pltpu.async_copy(p1v.at[idxd[bslot]], rbufs[bslot], gsems[bslot])

    def _gather_wait(bslot):
        pltpu.make_async_copy(
            p1v.at[idxd[bslot]], rbufs[bslot], gsems[bslot]).wait()

    def _scatter(bslot):
        pltpu.async_copy(
            rbufs[bslot], accs.at[ridd[bslot]], ssems[bslot], add=True)

    def _scatter_wait(bslot):
        pltpu.make_async_copy(
            rbufs[bslot], accs.at[ridd[bslot]], ssems[bslot]).wait()

    for fc in range(NFC):
        for bslot in range(2):
            _decode(bslot, bslot, fc, True)
            _gather(bslot)

        def pass_body(o, carry):
            for kk in range(NBUF):
                j = o * NBUF + kk
                bp = (kk + 2) % NBUF

                @pl.when(j >= 2)
                def _():
                    _scatter_wait(bp)

                @pl.when(j + 2 < nbat4)
                def _():
                    _decode(j + 2, bp, fc, True)
                    _gather(bp)

                _gather_wait(kk)
                _scatter(kk)
            return carry

        lax.fori_loop(0, nbat4 // 4, pass_body, 0)
        _scatter_wait(2)
        _scatter_wait(3)
        plsc.subcore_barrier()

        pltpu.sync_copy(
            accs.at[pl.ds(s * DSTRIPE, DSTRIPE)],
            s2a.at[pl.ds(c * RA + s * DSTRIPE, DSTRIPE), pl.ds(fc * L, L)],
        )
        _zero_stripe()
        plsc.subcore_barrier()

    def ones_fill(i, carry):
        rb0[i, :] = jnp.full((L,), 1.0, jnp.float32)
        return carry

    lax.fori_loop(0, K, ones_fill, 0)

    def _cscatter(bslot):
        pltpu.async_copy(rb0, accs.at[ridd[bslot]], ssems[bslot], add=True)

    def _cscatter_wait(bslot):
        pltpu.make_async_copy(rb0, accs.at[ridd[bslot]], ssems[bslot]).wait()

    def cnt_body(o, carry):
        for kk in range(NBUF):
            j = o * NBUF + kk

            @pl.when(j >= NBUF)
            def _():
                _cscatter_wait(kk)

            _decode(j, kk, 0, False)
            _cscatter(kk)
        return carry

    lax.fori_loop(0, nbat4 // 4, cnt_body, 0)
    for kk in range(NBUF):
        _cscatter_wait(kk)
    plsc.subcore_barrier()
    pltpu.sync_copy(
        accs.at[pl.ds(s * DSTRIPE, DSTRIPE)],
        s2c.at[pl.ds(c * RA + s * DSTRIPE, DSTRIPE)],
    )


_sc_call = functools.partial(
    pl.kernel,
    out_type=[
        jax.ShapeDtypeStruct((N * R, F), jnp.float32),
        jax.ShapeDtypeStruct((N * R, L), jnp.float32),
    ],
    mesh=plsc.VectorSubcoreMesh(core_axis_name="c", subcore_axis_name="s"),
    scratch_types=[
        pltpu.VMEM((PBUF,), jnp.int32),
        pltpu.VMEM((WN,), jnp.int32),
        pltpu.VMEM((WN,), jnp.int32),
        pltpu.VMEM((WN,), jnp.int32),
        pltpu.VMEM((K,), jnp.int32),
        pltpu.VMEM((K,), jnp.int32),
        pltpu.VMEM((K,), jnp.int32),
        pltpu.VMEM((K,), jnp.int32),
        pltpu.VMEM((K,), jnp.int32),
        pltpu.VMEM((K,), jnp.int32),
        pltpu.VMEM((K,), jnp.int32),
        pltpu.VMEM((K,), jnp.int32),
        pltpu.VMEM((K, L), jnp.float32),
        pltpu.VMEM((K, L), jnp.float32),
        pltpu.VMEM((K, L), jnp.float32),
        pltpu.VMEM((K, L), jnp.float32),
        pltpu.VMEM((ZROWS, L), jnp.float32),
        pltpu.VMEM_SHARED((ACC_ROWS, L), jnp.float32),
        pltpu.SemaphoreType.DMA,
        pltpu.SemaphoreType.DMA,
        pltpu.SemaphoreType.DMA,
        pltpu.SemaphoreType.DMA,
        pltpu.SemaphoreType.DMA,
        pltpu.SemaphoreType.DMA,
        pltpu.SemaphoreType.DMA,
        pltpu.SemaphoreType.DMA,
    ],
    compiler_params=pltpu.CompilerParams(
        use_tc_tiling_on_sc=False, needs_layout_passes=False),
)(_sc_body)


_FB = 400


def _fin_body(s_ref, c_ref, p2_ref, r3_ref, o_ref):
    sv = s_ref[...]
    cnt = c_ref[:, :, 0:1]
    p2 = p2_ref[...][:, None, :]
    r3 = r3_ref[...][None, :, :]
    agg = sv + cnt * (p2 + r3)
    o_ref[...] = jnp.tanh(agg).sum(axis=1)


def _finalize(s3, c3, P2, r3):
    return pl.pallas_call(
        _fin_body,
        grid=(N // _FB,),
        in_specs=[
            pl.BlockSpec((_FB, R, F), lambda i: (i, 0, 0)),
            pl.BlockSpec((_FB, R, L), lambda i: (i, 0, 0)),
            pl.BlockSpec((_FB, F), lambda i: (i, 0)),
            pl.BlockSpec((R, F), lambda i: (0, 0)),
        ],
        out_specs=pl.BlockSpec((_FB, F), lambda i: (i, 0)),
        out_shape=jax.ShapeDtypeStruct((N, F), jnp.float32),
    )(s3, c3, P2, r3)


def kernel(x, edge_index, edge_type, rel_emb, W, b):
    P1, P2, r3 = _proj(x, rel_emb, W, b.reshape(1, F))
    p1v = P1.reshape(N * NFC, L)
    src = edge_index[0].astype(jnp.int32)
    dst = edge_index[1].astype(jnp.int32)
    typ = edge_type.astype(jnp.int32)
    s2a, s2c = _sc_call(p1v, src, dst, typ)
    return _finalize(
        s2a.reshape(N, R, F), s2c.reshape(N, R, L), P2, r3
    )


# speedup vs baseline: 4.5122x; 1.0436x over previous
import functools

import jax
import jax.numpy as jnp
from jax import lax
from jax.experimental import pallas as pl
from jax.experimental.pallas import tpu as pltpu
from jax.experimental.pallas import tpu_sc as plsc

N = 10000
E = 320000
R = 16
F = 128
EMB = 64

NC = 2
NS = 16
L = 16

EP = E // NS
K = 128
NB = 160
EPAD = NB * K
NBUF = 4

NN = N // NC
RA = NN * R
DEAD = RA
ACC_ROWS = 80128
DSTRIPE = RA // NS
ZROWS = 250
NFC = F // L
WN = 2000
PK = 1 << 17
G = 2
GK = G * K
PBUF = EPAD + 1056


_XB = 1000


def _proj_body(x_ref, w_ref, re_ref, b_ref, p1_ref, p2_ref, r3_ref):
    xb = x_ref[...]
    p1_ref[...] = jnp.dot(xb, w_ref[0:F, :], preferred_element_type=jnp.float32)
    p2_ref[...] = jnp.dot(xb, w_ref[F:2 * F, :], preferred_element_type=jnp.float32)

    @pl.when(pl.program_id(0) == 0)
    def _():
        r3_ref[...] = (
            jnp.dot(re_ref[...], w_ref[2 * F:, :], preferred_element_type=jnp.float32)
            + b_ref[...]
        )


def _proj(x, rel_emb, W, b2d):
    return pl.pallas_call(
        _proj_body,
        grid=(N // _XB,),
        in_specs=[
            pl.BlockSpec((_XB, F), lambda i: (i, 0)),
            pl.BlockSpec((2 * F + EMB, F), lambda i: (0, 0)),
            pl.BlockSpec((R, EMB), lambda i: (0, 0)),
            pl.BlockSpec((1, F), lambda i: (0, 0)),
        ],
        out_specs=[
            pl.BlockSpec((_XB, F), lambda i: (i, 0)),
            pl.BlockSpec((_XB, F), lambda i: (i, 0)),
            pl.BlockSpec((R, F), lambda i: (0, 0)),
        ],
        out_shape=[
            jax.ShapeDtypeStruct((N, F), jnp.float32),
            jax.ShapeDtypeStruct((N, F), jnp.float32),
            jax.ShapeDtypeStruct((R, F), jnp.float32),
        ],
    )(x, W, rel_emb, b2d)


def _sc_body(p1v, srch, dsth, typh, s2a, s2c,
             packedb, dstw, typw, srcw,
             ridd0, ridd1, ridd2, ridd3, idxd0, idxd1, idxd2, idxd3,
             rb0, rb1, rb2, rb3, zb, accs,
             g0, g1, g2, g3, s0, s1, s2, s3):
    c = lax.axis_index("c")
    s = lax.axis_index("s")
    base = s * EP
    lo = c * NN
    rbufs = (rb0, rb1, rb2, rb3)
    ridd = (ridd0, ridd1, ridd2, ridd3)
    idxd = (idxd0, idxd1, idxd2, idxd3)
    gsems = (g0, g1, g2, g3)
    ssems = (s0, s1, s2, s3)

    off = jnp.int32(0)
    for w in range(EP // WN):
        wb = base + w * WN
        pltpu.async_copy(dsth.at[pl.ds(wb, WN)], dstw, g0)
        pltpu.async_copy(typh.at[pl.ds(wb, WN)], typw, g1)
        pltpu.async_copy(srch.at[pl.ds(wb, WN)], srcw, g2)
        pltpu.make_async_copy(dsth.at[pl.ds(wb, WN)], dstw, g0).wait()
        pltpu.make_async_copy(typh.at[pl.ds(wb, WN)], typw, g1).wait()
        pltpu.make_async_copy(srch.at[pl.ds(wb, WN)], srcw, g2).wait()

        def cstep(i, o):
            d = dstw[pl.ds(i * L, L)]
            t = typw[pl.ds(i * L, L)]
            sv = srcw[pl.ds(i * L, L)]
            dl = d - lo
            m = (dl >= 0) & (dl < NN)
            packed = sv * PK + (dl * R + t)
            cs = plsc.cumsum(jnp.where(m, jnp.int32(1), jnp.int32(0)))
            plsc.store_scatter(packedb, [o + cs - 1], packed, mask=m)
            return o + cs[L - 1]

        off = lax.fori_loop(0, WN // L, cstep, off)

    nbat = jnp.maximum((off + (GK - 1)) // GK, 4)
    nbat4 = ((nbat + 3) // 4) * 4
    npadg = (nbat4 * GK - off + (L - 1)) // L

    def pstep(i, carry):
        packedb[pl.ds(off + i * L, L)] = jnp.full((L,), DEAD, jnp.int32)
        return carry

    lax.fori_loop(0, npadg, pstep, 0)

    def zero_fill(i, carry):
        zb[i, :] = jnp.zeros((L,), jnp.float32)
        return carry

    lax.fori_loop(0, ZROWS, zero_fill, 0)

    def _zero_stripe():
        for kk in range(DSTRIPE // ZROWS):
            pltpu.sync_copy(
                zb, accs.at[pl.ds(s * DSTRIPE + kk * ZROWS, ZROWS)])
        pltpu.sync_copy(zb.at[pl.ds(0, 8)], accs.at[pl.ds(DEAD + s * 8, 8)])

    _zero_stripe()
    plsc.subcore_barrier()

    def _decode(j, bslot, fc, with_idx):
        def dk(i, carry):
            v = packedb[pl.ds(j * GK + i * L, L)]
            ridd[bslot][pl.ds(i * L, L)] = v & (PK - 1)
            if with_idx:
                idxd[bslot][pl.ds(i * L, L)] = (v // PK) * NFC + fc
            return carry

        lax.fori_loop(0, GK // L, dk, 0)

    def _gather(bslot):
        pltpu.async_copy(p1v.at[idxd[bslot]], rbufs[bslot], gsems[bslot])

    def _gather_wait(bslot):
        pltpu.make_async_copy(
            p1v.at[idxd[bslot]], rbufs[bslot], gsems[bslot]).wait()

    def _scatter(bslot):
        pltpu.async_copy(
            rbufs[bslot], accs.at[ridd[bslot]], ssems[bslot], add=True)

    def _scatter_wait(bslot):
        pltpu.make_async_copy(
            rbufs[bslot], accs.at[ridd[bslot]], ssems[bslot]).wait()

    for fc in range(NFC):
        for bslot in range(2):
            _decode(bslot, bslot, fc, True)
            _gather(bslot)

        def pass_body(o, carry):
            for kk in range(NBUF):
                j = o * NBUF + kk
                bp = (kk + 2) % NBUF

                @pl.when(j >= 2)
                def _():
                    _scatter_wait(bp)

                @pl.when(j + 2 < nbat4)
                def _():
                    _decode(j + 2, bp, fc, True)
                    _gather(bp)

                _gather_wait(kk)
                _scatter(kk)
            return carry

        lax.fori_loop(0, nbat4 // 4, pass_body, 0)
        _scatter_wait(2)
        _scatter_wait(3)
        plsc.subcore_barrier()

        pltpu.sync_copy(
            accs.at[pl.ds(s * DSTRIPE, DSTRIPE)],
            s2a.at[pl.ds(c * RA + s * DSTRIPE, DSTRIPE), pl.ds(fc * L, L)],
        )
        _zero_stripe()
        plsc.subcore_barrier()

    def ones_fill(i, carry):
        rb0[i, :] = jnp.full((L,), 1.0, jnp.float32)
        return carry

    lax.fori_loop(0, GK, ones_fill, 0)

    def _cscatter(bslot):
        pltpu.async_copy(rb0, accs.at[ridd[bslot]], ssems[bslot], add=True)

    def _cscatter_wait(bslot):
        pltpu.make_async_copy(rb0, accs.at[ridd[bslot]], ssems[bslot]).wait()

    def cnt_body(o, carry):
        for kk in range(NBUF):
            j = o * NBUF + kk

            @pl.when(j >= NBUF)
            def _():
                _cscatter_wait(kk)

            _decode(j, kk, 0, False)
            _cscatter(kk)
        return carry

    lax.fori_loop(0, nbat4 // 4, cnt_body, 0)
    for kk in range(NBUF):
        _cscatter_wait(kk)
    plsc.subcore_barrier()
    pltpu.sync_copy(
        accs.at[pl.ds(s * DSTRIPE, DSTRIPE)],
        s2c.at[pl.ds(c * RA + s * DSTRIPE, DSTRIPE)],
    )


_sc_call = functools.partial(
    pl.kernel,
    out_type=[
        jax.ShapeDtypeStruct((N * R, F), jnp.float32),
        jax.ShapeDtypeStruct((N * R, L), jnp.float32),
    ],
    mesh=plsc.VectorSubcoreMesh(core_axis_name="c", subcore_axis_name="s"),
    scratch_types=[
        pltpu.VMEM((PBUF,), jnp.int32),
        pltpu.VMEM((WN,), jnp.int32),
        pltpu.VMEM((WN,), jnp.int32),
        pltpu.VMEM((WN,), jnp.int32),
        pltpu.VMEM((GK,), jnp.int32),
        pltpu.VMEM((GK,), jnp.int32),
        pltpu.VMEM((GK,), jnp.int32),
        pltpu.VMEM((GK,), jnp.int32),
        pltpu.VMEM((GK,), jnp.int32),
        pltpu.VMEM((GK,), jnp.int32),
        pltpu.VMEM((GK,), jnp.int32),
        pltpu.VMEM((GK,), jnp.int32),
        pltpu.VMEM((GK, L), jnp.float32),
        pltpu.VMEM((GK, L), jnp.float32),
        pltpu.VMEM((GK, L), jnp.float32),
        pltpu.VMEM((GK, L), jnp.float32),
        pltpu.VMEM((ZROWS, L), jnp.float32),
        pltpu.VMEM_SHARED((ACC_ROWS, L), jnp.float32),
        pltpu.SemaphoreType.DMA,
        pltpu.SemaphoreType.DMA,
        pltpu.SemaphoreType.DMA,
        pltpu.SemaphoreType.DMA,
        pltpu.SemaphoreType.DMA,
        pltpu.SemaphoreType.DMA,
        pltpu.SemaphoreType.DMA,
        pltpu.SemaphoreType.DMA,
    ],
    compiler_params=pltpu.CompilerParams(
        use_tc_tiling_on_sc=False, needs_layout_passes=False),
)(_sc_body)


_FB = 400


def _fin_body(s_ref, c_ref, p2_ref, r3_ref, o_ref):
    sv = s_ref[...]
    cnt = c_ref[:, :, 0:1]
    p2 = p2_ref[...][:, None, :]
    r3 = r3_ref[...][None, :, :]
    agg = sv + cnt * (p2 + r3)
    o_ref[...] = jnp.tanh(agg).sum(axis=1)


def _finalize(s3, c3, P2, r3):
    return pl.pallas_call(
        _fin_body,
        grid=(N // _FB,),
        in_specs=[
            pl.BlockSpec((_FB, R, F), lambda i: (i, 0, 0)),
            pl.BlockSpec((_FB, R, L), lambda i: (i, 0, 0)),
            pl.BlockSpec((_FB, F), lambda i: (i, 0)),
            pl.BlockSpec((R, F), lambda i: (0, 0)),
        ],
        out_specs=pl.BlockSpec((_FB, F), lambda i: (i, 0)),
        out_shape=jax.ShapeDtypeStruct((N, F), jnp.float32),
    )(s3, c3, P2, r3)


def kernel(x, edge_index, edge_type, rel_emb, W, b):
    P1, P2, r3 = _proj(x, rel_emb, W, b.reshape(1, F))
    p1v = P1.reshape(N * NFC, L)
    src = edge_index[0].astype(jnp.int32)
    dst = edge_index[1].astype(jnp.int32)
    typ = edge_type.astype(jnp.int32)
    s2a, s2c = _sc_call(p1v, src, dst, typ)
    return _finalize(
        s2a.reshape(N, R, F), s2c.reshape(N, R, L), P2, r3
    )
